# baseline (device time: 224463 ns/iter reference)
import os

import jax
import jax.numpy as jnp
from jax import lax
from jax.experimental import pallas as pl
from jax.experimental.pallas import tpu as pltpu

_LOCAL_ONLY = bool(os.environ.get("LOCAL_ONLY"))

N_DEV = 4
CHUNKS = 8
SLOTS = 8


def kernel(x, w_mat):
    m_per, k = x.shape
    _, n = w_mat.shape
    n_per = n // N_DEV
    n_blk = n_per // CHUNKS
    grid = N_DEV * CHUNKS
    n_remote = (N_DEV - 1) * CHUNKS

    my = lax.axis_index("i")

    t16 = jnp.arange(2 * CHUNKS, dtype=jnp.int32)
    tgt = jnp.where(t16 % 2 == 0, (my + 1) % N_DEV, (my + 3) % N_DEV)
    qs = jnp.arange(CHUNKS, dtype=jnp.int32)
    cols = jnp.concatenate([
        tgt * CHUNKS + t16 // 2,
        ((my + 2) % N_DEV) * CHUNKS + qs,
        (my % N_DEV) * CHUNKS + qs,
    ])

    def body(cols_ref, x_ref, w_ref, out_ref,
             send_bufs, send_sems, recv_sem, local_sem):
        t = pl.program_id(0)
        my = lax.axis_index("i")
        jj = cols_ref[t] // CHUNKS
        q = lax.rem(cols_ref[t], CHUNKS)
        slot = lax.rem(t, SLOTS)

        @pl.when(t == 0)
        def _():
            barrier = pltpu.get_barrier_semaphore()
            for nbr in range(N_DEV):
                @pl.when(nbr != my)
                def _():
                    pl.semaphore_signal(
                        barrier, inc=1,
                        device_id=(nbr,),
                        device_id_type=pl.DeviceIdType.MESH,
                    )
            pl.semaphore_wait(barrier, N_DEV - 1)

        @pl.when((t >= SLOTS) & jnp.bool_(not _LOCAL_ONLY))
        def _():
            pltpu.make_async_remote_copy(
                src_ref=send_bufs.at[slot],
                dst_ref=send_bufs.at[slot],
                send_sem=send_sems.at[slot],
                recv_sem=recv_sem,
                device_id=(jj,),
                device_id_type=pl.DeviceIdType.MESH,
            ).wait_send()

        acc = jnp.dot(x_ref[:, :], w_ref[:, :],
                      preferred_element_type=jnp.float32)
        send_bufs[slot, :, :] = jnp.maximum(acc, 0.0)

        dst = out_ref.at[pl.ds(my * m_per, m_per), pl.ds(q * n_blk, n_blk)]

        @pl.when(t < n_remote)
        def _():
            if _LOCAL_ONLY:
                cp = pltpu.make_async_copy(send_bufs.at[slot], dst, local_sem)
                cp.start()
                cp.wait()
            else:
                pltpu.make_async_remote_copy(
                    src_ref=send_bufs.at[slot],
                    dst_ref=dst,
                    send_sem=send_sems.at[slot],
                    recv_sem=recv_sem,
                    device_id=(jj,),
                    device_id_type=pl.DeviceIdType.MESH,
                ).start()


        @pl.when(t >= n_remote)
        def _():
            cp = pltpu.make_async_copy(send_bufs.at[slot], dst, local_sem)
            cp.start()
            cp.wait()

        @pl.when((t == grid - 1) & jnp.bool_(not _LOCAL_ONLY))
        def _():
            recv = pltpu.make_async_remote_copy(
                src_ref=send_bufs.at[0],
                dst_ref=out_ref.at[pl.ds(0, m_per), pl.ds(0, n_blk)],
                send_sem=send_sems.at[0],
                recv_sem=recv_sem,
                device_id=(my,),
                device_id_type=pl.DeviceIdType.MESH,
            )
            for _ in range(n_remote):
                recv.wait_recv()

    grid_spec = pltpu.PrefetchScalarGridSpec(
        num_scalar_prefetch=1,
        grid=(grid,),
        in_specs=[
            pl.BlockSpec((m_per, k), lambda t, cols: (0, 0)),
            pl.BlockSpec((k, n_blk), lambda t, cols: (0, cols[t])),
        ],
        out_specs=pl.BlockSpec(memory_space=pl.ANY),
        scratch_shapes=[
            pltpu.VMEM((SLOTS, m_per, n_blk), jnp.float32),
            pltpu.SemaphoreType.DMA((SLOTS,)),
            pltpu.SemaphoreType.DMA,
            pltpu.SemaphoreType.DMA,
        ],
    )

    return pl.pallas_call(
        body,
        grid_spec=grid_spec,
        out_shape=jax.ShapeDtypeStruct((N_DEV * m_per, n_per), jnp.float32),
        compiler_params=pltpu.CompilerParams(
            collective_id=0,
            dimension_semantics=("arbitrary",),
            vmem_limit_bytes=60 * 1024 * 1024,
        ),
    )(cols, x, w_mat)


# device time: 144283 ns/iter; 1.5557x vs baseline; 1.5557x over previous
import jax
import jax.numpy as jnp
from jax import lax
from jax.experimental import pallas as pl
from jax.experimental.pallas import tpu as pltpu

N_DEV = 4
CHUNKS = 4
SLOTS = 4
STORE_BUFS = 3


def kernel(x, w_mat):
    m_per, k = x.shape
    _, n = w_mat.shape
    n_per = n // N_DEV
    n_blk = n_per // CHUNKS
    grid = N_DEV * CHUNKS
    n_remote = (N_DEV - 1) * CHUNKS
    alt = 2 * CHUNKS

    my = lax.axis_index("i")

    t16 = jnp.arange(alt, dtype=jnp.int32)
    tgt = jnp.where(t16 % 2 == 0, (my + 1) % N_DEV, (my + 3) % N_DEV)
    qs = jnp.arange(CHUNKS, dtype=jnp.int32)
    cols = jnp.concatenate([
        tgt * CHUNKS + t16 // 2,
        ((my + 2) % N_DEV) * CHUNKS + qs,
        (my % N_DEV) * CHUNKS + qs,
    ])

    sched = {}
    for c in range(CHUNKS):
        sched.setdefault(2 * c + 3, []).append(("rel", (0, c)))
        sched.setdefault(2 * c + 4, []).append(("rel", (2, c)))
        sched.setdefault(min(alt + 3 + c, grid - 1), []).append(
            ("rel", (1, c)))
        sched.setdefault(n_remote + c, []).append(("own", c))
    plan = {}
    u = 0
    for t in sorted(sched):
        plan[t] = []
        for op in sched[t]:
            plan[t].append((op, u))
            u += 1
    n_store = u

    def body(cols_ref, x_ref, w_ref, out_ref,
             send_bufs, recv_bufs, store_bufs,
             send_sems, recv_sems, store_sems):
        t = pl.program_id(0)
        my = lax.axis_index("i")
        col = cols_ref[t]
        jj = col // CHUNKS
        q = lax.rem(col, CHUNKS)
        slot = lax.rem(t, SLOTS)

        @pl.when(t == 0)
        def _():
            barrier = pltpu.get_barrier_semaphore()
            for nbr in range(N_DEV):
                @pl.when(nbr != my)
                def _():
                    pl.semaphore_signal(
                        barrier, inc=1,
                        device_id=(nbr,),
                        device_id_type=pl.DeviceIdType.MESH,
                    )
            pl.semaphore_wait(barrier, N_DEV - 1)

        @pl.when(t >= SLOTS)
        def _():
            pltpu.make_async_remote_copy(
                src_ref=send_bufs.at[slot],
                dst_ref=send_bufs.at[slot],
                send_sem=send_sems.at[slot],
                recv_sem=recv_sems.at[0],
                device_id=(jj,),
                device_id_type=pl.DeviceIdType.MESH,
            ).wait_send()

        acc = jnp.dot(x_ref[:, :], w_ref[:, :],
                      preferred_element_type=jnp.float32)
        y = jnp.maximum(acc, 0.0)

        @pl.when(t < n_remote)
        def _():
            send_bufs[slot, :, :] = y.astype(jnp.bfloat16)

        def send(d):
            pltpu.make_async_remote_copy(
                src_ref=send_bufs.at[slot],
                dst_ref=recv_bufs.at[(d - 1) * CHUNKS + q],
                send_sem=send_sems.at[slot],
                recv_sem=recv_sems.at[(d - 1) * CHUNKS + q],
                device_id=(jj,),
                device_id_type=pl.DeviceIdType.MESH,
            ).start()

        @pl.when((t < alt) & (lax.rem(t, 2) == 0))
        def _():
            send(1)

        @pl.when((t < alt) & (lax.rem(t, 2) == 1))
        def _():
            send(3)

        @pl.when((t >= alt) & (t < n_remote))
        def _():
            send(2)

        def chunk_wait(idx):
            pltpu.make_async_remote_copy(
                src_ref=recv_bufs.at[idx],
                dst_ref=recv_bufs.at[idx],
                send_sem=send_sems.at[0],
                recv_sem=recv_sems.at[idx],
                device_id=(my,),
                device_id_type=pl.DeviceIdType.MESH,
            ).wait_recv()

        def store_wait(sb):
            pltpu.make_async_copy(
                store_bufs.at[sb],
                out_ref.at[pl.ds(0, m_per), pl.ds(0, n_blk)],
                store_sems.at[sb],
            ).wait()

        for T, ops in plan.items():
            @pl.when(t == T)
            def _(ops=ops):
                for (kind, arg), uu in ops:
                    sb = uu % STORE_BUFS
                    if uu >= STORE_BUFS:
                        store_wait(sb)
                    if kind == "own":
                        qq = arg
                        store_bufs[sb, :, :] = y
                        rows = my * m_per
                    else:
                        rel, c = arg
                        qq = c
                        chunk_wait(rel * CHUNKS + c)
                        src_dev = lax.rem(my - (rel + 1) + N_DEV, N_DEV)
                        store_bufs[sb, :, :] = (
                            recv_bufs[rel * CHUNKS + c, :, :]
                            .astype(jnp.float32))
                        rows = src_dev * m_per
                    pltpu.make_async_copy(
                        store_bufs.at[sb],
                        out_ref.at[pl.ds(rows, m_per),
                                   pl.ds(qq * n_blk, n_blk)],
                        store_sems.at[sb],
                    ).start()

        @pl.when(t == grid - 1)
        def _():
            for uu in range(n_store - STORE_BUFS, n_store):
                store_wait(uu % STORE_BUFS)

    grid_spec = pltpu.PrefetchScalarGridSpec(
        num_scalar_prefetch=1,
        grid=(grid,),
        in_specs=[
            pl.BlockSpec((m_per, k), lambda t, cols: (0, 0)),
            pl.BlockSpec((k, n_blk), lambda t, cols: (0, cols[t])),
        ],
        out_specs=pl.BlockSpec(memory_space=pl.ANY),
        scratch_shapes=[
            pltpu.VMEM((SLOTS, m_per, n_blk), jnp.bfloat16),
            pltpu.VMEM((n_remote, m_per, n_blk), jnp.bfloat16),
            pltpu.VMEM((STORE_BUFS, m_per, n_blk), jnp.float32),
            pltpu.SemaphoreType.DMA((SLOTS,)),
            pltpu.SemaphoreType.DMA(((N_DEV - 1) * CHUNKS,)),
            pltpu.SemaphoreType.DMA((STORE_BUFS,)),
        ],
    )

    return pl.pallas_call(
        body,
        grid_spec=grid_spec,
        out_shape=jax.ShapeDtypeStruct((N_DEV * m_per, n_per), jnp.float32),
        compiler_params=pltpu.CompilerParams(
            collective_id=0,
            dimension_semantics=("arbitrary",),
            vmem_limit_bytes=60 * 1024 * 1024,
        ),
    )(cols, x, w_mat)
